# baseline (speedup 1.0000x reference)
import functools

import jax
import jax.numpy as jnp
from jax import lax
from jax.experimental import pallas as pl
from jax.experimental.pallas import tpu as pltpu
from jax.experimental.pallas import tpu_sc as plsc

_XRING = 4
_XDIST = 2
_GRING = 8
_UNROLL = 8


def _build(N, D, rows_per_worker, K):
    chunks = rows_per_worker // K
    nsteps = chunks // _UNROLL
    mesh = plsc.VectorSubcoreMesh(core_axis_name="c", subcore_axis_name="s")
    nc = mesh.num_cores

    def body(x_hbm, idx_hbm, tab_hbm, out_hbm, idx_all, *rest):
        bufs = rest[:_XRING]
        rbufs = rest[_XRING:_XRING + _GRING]
        si = rest[_XRING + _GRING]
        sxs = rest[_XRING + _GRING + 1:][:_XRING]
        sgs = rest[_XRING + _GRING + 1 + _XRING:][:_GRING]
        sos = rest[_XRING + _GRING + 1 + _XRING + _GRING:]

        wid = lax.axis_index("s") * nc + lax.axis_index("c")
        base = wid * rows_per_worker

        idx_load = pltpu.async_copy(
            idx_hbm.at[pl.ds(base, rows_per_worker)], idx_all, si)

        def start_gather(j, r, sem):
            pltpu.async_copy(tab_hbm.at[idx_all.at[pl.ds(j * K, K)]], r, sem)

        def start_x(j, buf, sem):
            pltpu.async_copy(x_hbm.at[pl.ds(base + j * K, K)], buf, sem)

        def wait_into(buf, sem):
            pltpu.make_async_copy(x_hbm.at[pl.ds(0, K)], buf, sem).wait()

        def start_out(j, buf, sem):
            pltpu.async_copy(buf, out_hbm.at[pl.ds(base + j * K, K)], sem)

        def wait_out(j, buf, sem):
            pltpu.make_async_copy(buf, out_hbm.at[pl.ds(base + j * K, K)], sem).wait()

        def add_chunk(buf, rbuf):
            half = D // 2
            for off in (0, half):
                @plsc.parallel_loop(0, K, unroll=1)
                def add_part(r, off=off):
                    for j in range(half // 16):
                        sl = pl.ds(off + j * 16, 16)
                        plsc.addupdate(buf.at[r, sl], rbuf[r, sl])

        for j in range(_XDIST):
            start_x(j, bufs[j], sxs[j])
        idx_load.wait()
        for j in range(_GRING):
            start_gather(j, rbufs[j], sgs[j])

        def step(q, _):
            j0 = _UNROLL * q
            for k in range(_UNROLL):
                j = j0 + k
                kx = k % _XRING
                buf, sx, so = bufs[kx], sxs[kx], sos[kx]
                rb, sg = rbufs[k % _GRING], sgs[k % _GRING]

                wait_into(rb, sg)
                wait_into(buf, sx)
                add_chunk(buf, rb)
                start_out(j, buf, so)

                @pl.when(j + _GRING < chunks)
                def _():
                    start_gather(j + _GRING, rb, sg)

                pk = (kx + _XDIST) % _XRING

                @pl.when(j >= _XRING - _XDIST)
                def _():
                    wait_out(j - (_XRING - _XDIST), bufs[pk], sos[pk])

                @pl.when(j + _XDIST < chunks)
                def _():
                    start_x(j + _XDIST, bufs[pk], sxs[pk])

            return 0

        lax.fori_loop(0, nsteps, step, 0)

        for j in range(chunks - (_XRING - _XDIST), chunks):
            wait_out(j, bufs[j % _XRING], sos[j % _XRING])

    return pl.kernel(
        body,
        out_type=jax.ShapeDtypeStruct((N, D), jnp.float32),
        mesh=mesh,
        scratch_types=[pltpu.VMEM((rows_per_worker,), jnp.int32)]
        + [pltpu.VMEM((K, D), jnp.float32)] * (_XRING + _GRING)
        + [pltpu.SemaphoreType.DMA] * (1 + _XRING + _GRING + _XRING),
    )


@jax.jit
def kernel(x, positions, pos_emb):
    B, S, D = x.shape
    N = B * S
    nw = 32
    rows_per_worker = N // nw
    fn = _build(N, D, rows_per_worker, K=8)
    out = fn(x.reshape(N, D), positions.reshape(N), pos_emb)
    return out.reshape(B, S, D)


# speedup vs baseline: 1.0609x; 1.0609x over previous
import functools

import jax
import jax.numpy as jnp
from jax import lax
from jax.experimental import pallas as pl
from jax.experimental.pallas import tpu as pltpu
from jax.experimental.pallas import tpu_sc as plsc

_XRING = 8
_XDIST = 5
_GRING = 4
_UNROLL = 8


def _build(N, D, rows_per_worker, K):
    chunks = rows_per_worker // K
    nsteps = chunks // _UNROLL
    mesh = plsc.VectorSubcoreMesh(core_axis_name="c", subcore_axis_name="s")
    nc = mesh.num_cores

    def body(x_hbm, idx_hbm, tab_hbm, out_hbm, idx_all, *rest):
        bufs = rest[:_XRING]
        rbufs = rest[_XRING:_XRING + _GRING]
        si = rest[_XRING + _GRING]
        sxs = rest[_XRING + _GRING + 1:][:_XRING]
        sgs = rest[_XRING + _GRING + 1 + _XRING:][:_GRING]
        sos = rest[_XRING + _GRING + 1 + _XRING + _GRING:]

        wid = lax.axis_index("s") * nc + lax.axis_index("c")
        base = wid * rows_per_worker

        idx_load = pltpu.async_copy(
            idx_hbm.at[pl.ds(base, rows_per_worker)], idx_all, si)

        def start_gather(j, r, sem):
            pltpu.async_copy(tab_hbm.at[idx_all.at[pl.ds(j * K, K)]], r, sem)

        def start_x(j, buf, sem):
            pltpu.async_copy(x_hbm.at[pl.ds(base + j * K, K)], buf, sem)

        def wait_into(buf, sem):
            pltpu.make_async_copy(x_hbm.at[pl.ds(0, K)], buf, sem).wait()

        def start_out(j, buf, sem):
            pltpu.async_copy(buf, out_hbm.at[pl.ds(base + j * K, K)], sem)

        def wait_out(j, buf, sem):
            pltpu.make_async_copy(buf, out_hbm.at[pl.ds(base + j * K, K)], sem).wait()

        def add_chunk(buf, rbuf):
            half = D // 2
            for off in (0, half):
                @plsc.parallel_loop(0, K, unroll=1)
                def add_part(r, off=off):
                    for j in range(half // 16):
                        sl = pl.ds(off + j * 16, 16)
                        plsc.addupdate(buf.at[r, sl], rbuf[r, sl])

        for j in range(_XDIST):
            start_x(j, bufs[j], sxs[j])
        idx_load.wait()
        for j in range(_GRING):
            start_gather(j, rbufs[j], sgs[j])

        def step(q, _):
            j0 = _UNROLL * q
            for k in range(_UNROLL):
                j = j0 + k
                kx = k % _XRING
                buf, sx, so = bufs[kx], sxs[kx], sos[kx]
                rb, sg = rbufs[k % _GRING], sgs[k % _GRING]

                wait_into(rb, sg)
                wait_into(buf, sx)
                add_chunk(buf, rb)
                start_out(j, buf, so)

                @pl.when(j + _GRING < chunks)
                def _():
                    start_gather(j + _GRING, rb, sg)

                pk = (kx + _XDIST) % _XRING

                @pl.when(j >= _XRING - _XDIST)
                def _():
                    wait_out(j - (_XRING - _XDIST), bufs[pk], sos[pk])

                @pl.when(j + _XDIST < chunks)
                def _():
                    start_x(j + _XDIST, bufs[pk], sxs[pk])

            return 0

        lax.fori_loop(0, nsteps, step, 0)

        for j in range(chunks - (_XRING - _XDIST), chunks):
            wait_out(j, bufs[j % _XRING], sos[j % _XRING])

    return pl.kernel(
        body,
        out_type=jax.ShapeDtypeStruct((N, D), jnp.float32),
        mesh=mesh,
        scratch_types=[pltpu.VMEM((rows_per_worker,), jnp.int32)]
        + [pltpu.VMEM((K, D), jnp.float32)] * (_XRING + _GRING)
        + [pltpu.SemaphoreType.DMA] * (1 + _XRING + _GRING + _XRING),
    )


@jax.jit
def kernel(x, positions, pos_emb):
    B, S, D = x.shape
    N = B * S
    nw = 32
    rows_per_worker = N // nw
    fn = _build(N, D, rows_per_worker, K=8)
    out = fn(x.reshape(N, D), positions.reshape(N), pos_emb)
    return out.reshape(B, S, D)


# speedup vs baseline: 1.0656x; 1.0044x over previous
import functools

import jax
import jax.numpy as jnp
from jax import lax
from jax.experimental import pallas as pl
from jax.experimental.pallas import tpu as pltpu
from jax.experimental.pallas import tpu_sc as plsc

_XRING = 8
_XDIST = 4
_GRING = 4
_UNROLL = 8


def _build(N, D, rows_per_worker, K):
    chunks = rows_per_worker // K
    nsteps = chunks // _UNROLL
    mesh = plsc.VectorSubcoreMesh(core_axis_name="c", subcore_axis_name="s")
    nc = mesh.num_cores

    def body(x_hbm, idx_hbm, tab_hbm, out_hbm, idx_all, *rest):
        bufs = rest[:_XRING]
        rbufs = rest[_XRING:_XRING + _GRING]
        si = rest[_XRING + _GRING]
        sxs = rest[_XRING + _GRING + 1:][:_XRING]
        sgs = rest[_XRING + _GRING + 1 + _XRING:][:_GRING]
        sos = rest[_XRING + _GRING + 1 + _XRING + _GRING:]

        wid = lax.axis_index("s") * nc + lax.axis_index("c")
        base = wid * rows_per_worker

        idx_load = pltpu.async_copy(
            idx_hbm.at[pl.ds(base, rows_per_worker)], idx_all, si)

        def start_gather(j, r, sem):
            pltpu.async_copy(tab_hbm.at[idx_all.at[pl.ds(j * K, K)]], r, sem)

        def start_x(j, buf, sem):
            pltpu.async_copy(x_hbm.at[pl.ds(base + j * K, K)], buf, sem)

        def wait_into(buf, sem):
            pltpu.make_async_copy(x_hbm.at[pl.ds(0, K)], buf, sem).wait()

        def start_out(j, buf, sem):
            pltpu.async_copy(buf, out_hbm.at[pl.ds(base + j * K, K)], sem)

        def wait_out(j, buf, sem):
            pltpu.make_async_copy(buf, out_hbm.at[pl.ds(base + j * K, K)], sem).wait()

        def add_chunk(buf, rbuf):
            half = D // 2
            for off in (0, half):
                @plsc.parallel_loop(0, K, unroll=1)
                def add_part(r, off=off):
                    for j in range(half // 16):
                        sl = pl.ds(off + j * 16, 16)
                        plsc.addupdate(buf.at[r, sl], rbuf[r, sl])

        for j in range(_XDIST):
            start_x(j, bufs[j], sxs[j])
        idx_load.wait()
        for j in range(_GRING):
            start_gather(j, rbufs[j], sgs[j])

        def step(q, _):
            j0 = _UNROLL * q
            for k in range(_UNROLL):
                j = j0 + k
                kx = k % _XRING
                buf, sx, so = bufs[kx], sxs[kx], sos[kx]
                rb, sg = rbufs[k % _GRING], sgs[k % _GRING]

                wait_into(rb, sg)
                wait_into(buf, sx)
                add_chunk(buf, rb)
                start_out(j, buf, so)

                @pl.when(j + _GRING < chunks)
                def _():
                    start_gather(j + _GRING, rb, sg)

                pk = (kx + _XDIST) % _XRING

                @pl.when(j >= _XRING - _XDIST)
                def _():
                    wait_out(j - (_XRING - _XDIST), bufs[pk], sos[pk])

                @pl.when(j + _XDIST < chunks)
                def _():
                    start_x(j + _XDIST, bufs[pk], sxs[pk])

            return 0

        lax.fori_loop(0, nsteps, step, 0)

        for j in range(chunks - (_XRING - _XDIST), chunks):
            wait_out(j, bufs[j % _XRING], sos[j % _XRING])

    return pl.kernel(
        body,
        out_type=jax.ShapeDtypeStruct((N, D), jnp.float32),
        mesh=mesh,
        scratch_types=[pltpu.VMEM((rows_per_worker,), jnp.int32)]
        + [pltpu.VMEM((K, D), jnp.float32)] * (_XRING + _GRING)
        + [pltpu.SemaphoreType.DMA] * (1 + _XRING + _GRING + _XRING),
    )


@jax.jit
def kernel(x, positions, pos_emb):
    B, S, D = x.shape
    N = B * S
    nw = 32
    rows_per_worker = N // nw
    fn = _build(N, D, rows_per_worker, K=8)
    out = fn(x.reshape(N, D), positions.reshape(N), pos_emb)
    return out.reshape(B, S, D)
